# baseline (speedup 1.0000x reference)
import functools

import jax
import jax.numpy as jnp
from jax.experimental import pallas as pl
from jax.experimental.pallas import tpu as pltpu

_RB = 200
_CB = 500
_MAXIT = 24


def _emb_body(f_ref, w0_ref, w1_ref, o_ref):
    h = jnp.maximum(f_ref[...] * w0_ref[...], 0.0) * w1_ref[...]
    s2 = jnp.sum(h * h, axis=1, keepdims=True)
    nrm = jnp.maximum(jnp.sqrt(s2), 1e-12)
    o_ref[...] = h / nrm


def _slices(n):
    out = []
    st = 0
    while st < n:
        out.append((st, min(1024, n - st)))
        st += 1024
    return out


def _sim_body(nch, n, emb_r_ref, emb3_ref, kf_ref, o_ref):
    j = pl.program_id(1)

    @pl.when(j == 0)
    def _compute():
        er = emb_r_ref[...]
        for cc in range(nch):
            ec = emb3_ref[cc]
            sim = jax.lax.dot_general(
                er, ec, (((1,), (1,)), ((), ())),
                preferred_element_type=jnp.float32)
            o_ref[:, cc * _CB:(cc + 1) * _CB] = sim

    @pl.when(j == 1)
    def _finish():
        kp1 = kf_ref[0, 0]
        logkp1 = jnp.log(kp1)
        sls = _slices(n)

        def cond(st):
            i, lo, c_lo, hi, c_hi = st
            return (i < _MAXIT) & ~jnp.all(
                (c_lo == kp1) | (c_lo - c_hi <= 2.0))

        def body(st):
            i, lo, c_lo, hi, c_hi = st
            done = (c_lo == kp1) | (c_lo - c_hi <= 2.0)
            w = hi - lo
            lcl = jnp.log(c_lo)
            lch = jnp.log(jnp.maximum(c_hi, 0.3))
            t = lo + w * (lcl - logkp1) / (lcl - lch)
            t = jnp.minimum(jnp.maximum(t, lo + 0.02 * w), hi - 0.02 * w)
            t = jnp.where(w < 1e-7, lo + 0.5 * w, t)
            cnt = jnp.zeros((_RB, 1), jnp.float32)
            for st_, w_ in sls:
                v = o_ref[:, st_:st_ + w_]
                mask = (v >= t).astype(jnp.float32)
                cnt += jax.lax.dot_general(
                    mask, jnp.ones((w_, 128), jnp.float32),
                    (((1,), (0,)), ((), ())),
                    preferred_element_type=jnp.float32)[:, 0:1]
            upd = ~done
            ge = upd & (cnt >= kp1)
            lt = upd & (cnt < kp1)
            return (i + 1, jnp.where(ge, t, lo), jnp.where(ge, cnt, c_lo),
                    jnp.where(lt, t, hi), jnp.where(lt, cnt, c_hi))

        st0 = (jnp.asarray(0, jnp.int32),
               jnp.full((_RB, 1), -1.01, jnp.float32),
               jnp.full((_RB, 1), float(n), jnp.float32),
               jnp.full((_RB, 1), 1.01, jnp.float32),
               jnp.zeros((_RB, 1), jnp.float32))
        _, lo, c_lo, hi, c_hi = jax.lax.while_loop(cond, body, st0)

        u1 = jnp.full((_RB, 1), -2.0, jnp.float32)
        b1 = jnp.full((_RB, 1), 2.0, jnp.float32)
        for st_, w_ in sls:
            v = o_ref[:, st_:st_ + w_]
            u1 = jnp.maximum(u1, jnp.max(
                jnp.where(v < hi, v, -2.0), axis=1, keepdims=True))
            b1 = jnp.minimum(b1, jnp.min(
                jnp.where(v >= lo, v, 2.0), axis=1, keepdims=True))
        pos = kp1 - c_hi
        m = c_lo - c_hi
        thr_u = jnp.where(pos <= 1.0, u1, b1)
        thr = jnp.where(c_lo == kp1, lo, jnp.where(m <= 2.0, thr_u, lo))

        for st_, w_ in sls:
            v = o_ref[:, st_:st_ + w_]
            o_ref[:, st_:st_ + w_] = jnp.where(
                v >= thr, jnp.maximum(v, 0.0), 0.0)


def kernel(features, w0, w1, k):
    n, d = features.shape
    assert n % _RB == 0 and n % _CB == 0
    nrb = n // _RB
    nch = n // _CB

    emb = pl.pallas_call(
        _emb_body,
        grid=(nrb,),
        in_specs=[pl.BlockSpec((_RB, d), lambda r: (r, 0)),
                  pl.BlockSpec((1, d), lambda r: (0, 0)),
                  pl.BlockSpec((1, d), lambda r: (0, 0))],
        out_specs=pl.BlockSpec((_RB, d), lambda r: (r, 0)),
        out_shape=jax.ShapeDtypeStruct((n, d), jnp.float32),
    )(features, w0.reshape(1, d), w1.reshape(1, d))

    emb3 = emb.reshape(nch, _CB, d)
    kf = jnp.asarray(k, jnp.float32).reshape(1, 1) + 1.0

    out = pl.pallas_call(
        functools.partial(_sim_body, nch, n),
        grid=(nrb, 2),
        in_specs=[pl.BlockSpec((_RB, d), lambda r, j: (r, 0)),
                  pl.BlockSpec((nch, _CB, d), lambda r, j: (0, 0, 0)),
                  pl.BlockSpec((1, 1), lambda r, j: (0, 0))],
        out_specs=pl.BlockSpec((_RB, n), lambda r, j: (r, 0)),
        out_shape=jax.ShapeDtypeStruct((n, n), jnp.float32),
        compiler_params=pltpu.CompilerParams(
            dimension_semantics=("parallel", "arbitrary")),
    )(emb, emb3, kf)
    return out


# speedup vs baseline: 1.3237x; 1.3237x over previous
import functools

import jax
import jax.numpy as jnp
from jax.experimental import pallas as pl
from jax.experimental.pallas import tpu as pltpu

_RB = 200
_CB = 500
_MAXIT = 24


def _emb_body(f_ref, w0_ref, w1_ref, o_ref):
    h = jnp.maximum(f_ref[...] * w0_ref[...], 0.0) * w1_ref[...]
    s2 = jnp.sum(h * h, axis=1, keepdims=True)
    nrm = jnp.maximum(jnp.sqrt(s2), 1e-12)
    o_ref[...] = h / nrm


def _slices(n):
    out = []
    st = 0
    while st < n:
        out.append((st, min(1024, n - st)))
        st += 1024
    return out


def _sim_body(nch, n, emb_r_ref, emb3_ref, kf_ref, o_ref):
    j = pl.program_id(1)

    @pl.when(j == 0)
    def _compute():
        er = emb_r_ref[...]
        for cc in range(nch):
            ec = emb3_ref[cc]
            sim = jax.lax.dot_general(
                er, ec, (((1,), (1,)), ((), ())),
                preferred_element_type=jnp.float32)
            o_ref[:, cc * _CB:(cc + 1) * _CB] = sim

    @pl.when(j == 1)
    def _finish():
        kp1 = kf_ref[0, 0]
        logkp1 = jnp.log(kp1)
        sls = _slices(n)

        def cond(st):
            i, lo, c_lo, hi, c_hi = st
            return (i < _MAXIT) & ~jnp.all(
                (c_lo == kp1) | (c_lo - c_hi <= 2.0))

        def body(st):
            i, lo, c_lo, hi, c_hi = st
            done = (c_lo == kp1) | (c_lo - c_hi <= 2.0)
            w = hi - lo
            lcl = jnp.log(c_lo)
            lch = jnp.log(jnp.maximum(c_hi, 0.3))
            t = lo + w * (lcl - logkp1) / (lcl - lch)
            t = jnp.minimum(jnp.maximum(t, lo + 0.02 * w), hi - 0.02 * w)
            t = jnp.where(w < 1e-7, lo + 0.5 * w, t)
            cnt = jnp.zeros((_RB, 1), jnp.float32)
            for st_, w_ in sls:
                v = o_ref[:, st_:st_ + w_]
                cnt += jnp.sum((v >= t).astype(jnp.float32),
                               axis=1, keepdims=True)
            upd = ~done
            ge = upd & (cnt >= kp1)
            lt = upd & (cnt < kp1)
            return (i + 1, jnp.where(ge, t, lo), jnp.where(ge, cnt, c_lo),
                    jnp.where(lt, t, hi), jnp.where(lt, cnt, c_hi))

        st0 = (jnp.asarray(0, jnp.int32),
               jnp.full((_RB, 1), -1.01, jnp.float32),
               jnp.full((_RB, 1), float(n), jnp.float32),
               jnp.full((_RB, 1), 1.01, jnp.float32),
               jnp.zeros((_RB, 1), jnp.float32))
        _, lo, c_lo, hi, c_hi = jax.lax.while_loop(cond, body, st0)

        u1 = jnp.full((_RB, 1), -2.0, jnp.float32)
        b1 = jnp.full((_RB, 1), 2.0, jnp.float32)
        for st_, w_ in sls:
            v = o_ref[:, st_:st_ + w_]
            u1 = jnp.maximum(u1, jnp.max(
                jnp.where(v < hi, v, -2.0), axis=1, keepdims=True))
            b1 = jnp.minimum(b1, jnp.min(
                jnp.where(v >= lo, v, 2.0), axis=1, keepdims=True))
        pos = kp1 - c_hi
        m = c_lo - c_hi
        thr_u = jnp.where(pos <= 1.0, u1, b1)
        thr = jnp.where(c_lo == kp1, lo, jnp.where(m <= 2.0, thr_u, lo))

        for st_, w_ in sls:
            v = o_ref[:, st_:st_ + w_]
            o_ref[:, st_:st_ + w_] = jnp.where(
                v >= thr, jnp.maximum(v, 0.0), 0.0)


def kernel(features, w0, w1, k):
    n, d = features.shape
    assert n % _RB == 0 and n % _CB == 0
    nrb = n // _RB
    nch = n // _CB

    emb = pl.pallas_call(
        _emb_body,
        grid=(nrb,),
        in_specs=[pl.BlockSpec((_RB, d), lambda r: (r, 0)),
                  pl.BlockSpec((1, d), lambda r: (0, 0)),
                  pl.BlockSpec((1, d), lambda r: (0, 0))],
        out_specs=pl.BlockSpec((_RB, d), lambda r: (r, 0)),
        out_shape=jax.ShapeDtypeStruct((n, d), jnp.float32),
    )(features, w0.reshape(1, d), w1.reshape(1, d))

    emb3 = emb.reshape(nch, _CB, d)
    kf = jnp.asarray(k, jnp.float32).reshape(1, 1) + 1.0

    out = pl.pallas_call(
        functools.partial(_sim_body, nch, n),
        grid=(nrb, 2),
        in_specs=[pl.BlockSpec((_RB, d), lambda r, j: (r, 0)),
                  pl.BlockSpec((nch, _CB, d), lambda r, j: (0, 0, 0)),
                  pl.BlockSpec((1, 1), lambda r, j: (0, 0))],
        out_specs=pl.BlockSpec((_RB, n), lambda r, j: (r, 0)),
        out_shape=jax.ShapeDtypeStruct((n, n), jnp.float32),
        compiler_params=pltpu.CompilerParams(
            dimension_semantics=("parallel", "arbitrary")),
    )(emb, emb3, kf)
    return out
